# baseline (speedup 1.0000x reference)
import functools

import jax
import jax.numpy as jnp
from jax import lax
from jax.experimental import pallas as pl
from jax.experimental.pallas import tpu as pltpu, tpu_sc as plsc

MAX_GOALS = 16384
DEPTH = 16
D = 256
G = 4096

NC = 2
NS = 16
L = 16
NW = NC * NS
QPW = G // NW
C = 8
NCHUNK = QPW // C
NPAIR = NCHUNK // 2
DV = D // L
PIECE = 32
BUFROWS = 160
RIDX = QPW * DEPTH + 2 * L

_mesh = plsc.VectorSubcoreMesh(
    core_axis_name="c", subcore_axis_name="s", num_cores=NC, num_subcores=NS
)


@functools.partial(
    pl.kernel,
    out_type=(
        jax.ShapeDtypeStruct((G, D), jnp.float32),
        jax.ShapeDtypeStruct((G,), jnp.float32),
    ),
    mesh=_mesh,
    compiler_params=pltpu.CompilerParams(needs_layout_passes=False),
    scratch_types=[
        pltpu.VMEM((QPW,), jnp.int32),
        pltpu.VMEM((QPW,), jnp.int32),
        pltpu.VMEM((QPW + 2 * L,), jnp.int32),
        pltpu.VMEM((RIDX,), jnp.int32),
        pltpu.VMEM((BUFROWS, D), jnp.float32),
        pltpu.VMEM((BUFROWS, D), jnp.float32),
        pltpu.VMEM((C, D), jnp.float32),
        pltpu.VMEM((QPW,), jnp.float32),
        pltpu.SemaphoreType.DMA,
        pltpu.SemaphoreType.DMA,
    ],
)
def _tracker(gidx_hbm, cnt_hbm, rows_hbm, sum_hbm, cn_hbm,
             gidx_v, cnt_v, off_v, ridx_v, buf0_v, buf1_v, out_v, cn_v,
             sem0, sem1):
    wid = lax.axis_index("s") * NC + lax.axis_index("c")
    base = wid * QPW

    pltpu.sync_copy(gidx_hbm.at[pl.ds(base, QPW)], gidx_v)
    for t in range(QPW // L):
        g = gidx_v[pl.ds(t * L, L)]
        gidx_v[pl.ds(t * L, L)] = jnp.clip(g, 0, MAX_GOALS - 1)

    pltpu.async_copy(cnt_hbm.at[gidx_v], cnt_v, sem0).wait()

    for t in range(QPW // L):
        nv = jnp.minimum(cnt_v[pl.ds(t * L, L)], DEPTH).astype(jnp.float32)
        cn_v[pl.ds(t * L, L)] = nv * (1.0 / DEPTH)
    pltpu.sync_copy(cn_v, cn_hbm.at[pl.ds(base, QPW)])

    zero16 = jnp.zeros((L,), jnp.int32)
    for t in range(RIDX // L):
        ridx_v[pl.ds(t * L, L)] = zero16

    run = jnp.int32(0)
    for t in range(QPW // L):
        c16 = jnp.minimum(cnt_v[pl.ds(t * L, L)], DEPTH)
        incl = plsc.cumsum(c16)
        off16 = incl - c16 + run
        off_v[pl.ds(t * L, L)] = off16
        g16 = gidx_v[pl.ds(t * L, L)] * DEPTH
        for j in range(DEPTH):
            plsc.store_scatter(ridx_v, [off16 + j], g16 + j, mask=j < c16)
        run = run + incl[L - 1]
    for t in range(QPW // L, QPW // L + 2):
        off_v[pl.ds(t * L, L)] = jnp.full((L,), run)

    bufs = (buf0_v, buf1_v)
    sems = (sem0, sem1)

    def chunk_meta(off16, nxt16, b):
        s = off16[8 * b]
        e = off16[8] if b == 0 else nxt16[0]
        cs8 = pl.multiple_of(s & -8, 8)
        npc = (e - cs8 + (PIECE - 1)) >> 5
        return cs8, npc

    def issue(cs8, npc, b):
        def body(t, carry):
            pltpu.async_copy(
                rows_hbm.at[ridx_v.at[pl.ds(cs8 + t * PIECE, PIECE)]],
                bufs[b].at[pl.ds(t * PIECE, PIECE)],
                sems[b],
            )
            return carry
        lax.fori_loop(0, npc, body, 0)

    def drain(cs8, npc, b):
        def body(t, carry):
            pltpu.make_async_copy(
                rows_hbm.at[ridx_v.at[pl.ds(cs8 + t * PIECE, PIECE)]],
                bufs[b].at[pl.ds(t * PIECE, PIECE)],
                sems[b],
            ).wait()
            return carry
        lax.fori_loop(0, npc, body, 0)

    off16_0 = off_v[pl.ds(0, L)]
    nxt16_0 = off_v[pl.ds(L, L)]
    for b in range(2):
        cs8, npc = chunk_meta(off16_0, nxt16_0, b)
        issue(cs8, npc, b)

    def pair_body(cp, carry):
        off16 = off_v[pl.ds(cp * L, L)]
        nxt16 = off_v[pl.ds(cp * L + L, L)]
        nxt2 = off_v[pl.ds(cp * L + 2 * L, L)]
        n16 = jnp.minimum(cnt_v[pl.ds(cp * L, L)], DEPTH)
        inv16 = 1.0 / jnp.maximum(n16.astype(jnp.float32), 1.0)

        for b in range(2):
            cs8, npc = chunk_meta(off16, nxt16, b)
            drain(cs8, npc, b)
            buf_v = bufs[b]

            for q in range(C):
                n_s = n16[b * C + q]
                loff = off16[b * C + q] - cs8
                inv_b = jnp.full((L,), inv16[b * C + q])

                def row_body(j, acc, loff=loff, buf_v=buf_v):
                    return tuple(
                        acc[v] + buf_v[loff + j, pl.ds(v * L, L)]
                        for v in range(DV)
                    )

                acc0 = tuple(jnp.zeros((L,), jnp.float32) for _ in range(DV))
                acc = lax.fori_loop(0, n_s, row_body, acc0)
                for v in range(DV):
                    out_v[q, pl.ds(v * L, L)] = acc[v] * inv_b

            pltpu.sync_copy(
                out_v, sum_hbm.at[pl.ds(base + (cp * 2 + b) * C, C)]
            )

            @pl.when(cp < NPAIR - 1)
            def _(b=b, nxt16=nxt16, nxt2=nxt2):
                cs8n, npcn = chunk_meta(nxt16, nxt2, b)
                issue(cs8n, npcn, b)

        return carry

    lax.fori_loop(0, NPAIR, pair_body, 0)


def kernel(goal_indices, failed_angles, failed_count):
    rows = failed_angles.reshape(MAX_GOALS * DEPTH, D)
    summary, count_norm = _tracker(goal_indices, failed_count, rows)
    return summary, count_norm
